# baseline (speedup 1.0000x reference)
import functools

import jax
import jax.numpy as jnp
from jax.experimental import pallas as pl
from jax.experimental.pallas import tpu as pltpu

D_MODEL = 2048
D_OUT = 2048
E = 8
R = 64
ER = E * R
SCALING = 128.0 / 64.0

TILE = 512
RW_PAD = 128
H_OFF = RW_PAD
B_OFF = RW_PAD + ER
WCAT_ROWS = RW_PAD + ER + D_OUT


def _pack_kernel(rw_ref, a_ref, w_ref, lb_ref, wcat_ref, bcat_ref):
    wcat_ref[0:E, :] = jnp.transpose(rw_ref[...], (1, 0)).astype(jnp.bfloat16)
    wcat_ref[E:RW_PAD, :] = jnp.zeros((RW_PAD - E, D_MODEL), jnp.bfloat16)
    wcat_ref[H_OFF:B_OFF, :] = a_ref[...].astype(jnp.bfloat16)
    wcat_ref[B_OFF:, :] = w_ref[...].astype(jnp.bfloat16)
    lb = lb_ref[...]
    bcat_ref[...] = jnp.transpose(lb, (0, 2, 1)).astype(
        jnp.bfloat16).reshape(ER, D_OUT)


def _fused_kernel(xf_ref, wcat_ref, b_ref, bcat_ref, o_ref):
    xb = xf_ref[...].astype(jnp.bfloat16)

    big = jax.lax.dot_general(
        xb, wcat_ref[...], (((1,), (1,)), ((), ())),
        preferred_element_type=jnp.float32)

    logits = big[:, :E]
    h = big[:, H_OFF:B_OFF]
    base = big[:, B_OFF:]

    m = jnp.max(logits, axis=1, keepdims=True)
    p = jnp.exp(logits - m)
    eidx = jax.lax.broadcasted_iota(jnp.int32, (TILE, E), 1)

    v1 = jnp.max(p, axis=1, keepdims=True)
    i1 = jnp.min(jnp.where(p == v1, eidx, E), axis=1, keepdims=True)
    p2 = jnp.where(eidx == i1, -1.0, p)
    v2 = jnp.max(p2, axis=1, keepdims=True)
    i2 = jnp.min(jnp.where(p2 == v2, eidx, E), axis=1, keepdims=True)

    denom = v1 + v2
    g1 = (v1 / denom) * SCALING
    g2 = (v2 / denom) * SCALING

    ecol = jax.lax.broadcasted_iota(jnp.int32, (TILE, ER), 1) // R
    gates = jnp.where(ecol == i1, g1, 0.0) + jnp.where(ecol == i2, g2, 0.0)
    hw = (h * gates).astype(jnp.bfloat16)

    moe = jax.lax.dot_general(
        hw, bcat_ref[...], (((1,), (0,)), ((), ())),
        preferred_element_type=jnp.float32)

    o_ref[...] = base + moe + b_ref[...]


@functools.partial(jax.jit, static_argnames=())
def kernel(x, W_base, b_base, router_w, lora_A, lora_B):
    B, S, D = x.shape
    N = B * S
    xf = x.reshape(N, D)

    w_cat, b_cat = pl.pallas_call(
        _pack_kernel,
        out_shape=(
            jax.ShapeDtypeStruct((WCAT_ROWS, D_MODEL), jnp.bfloat16),
            jax.ShapeDtypeStruct((ER, D_OUT), jnp.bfloat16),
        ),
    )(router_w, lora_A.reshape(ER, D_MODEL), W_base, lora_B)
    b2 = b_base.reshape(1, D_OUT)

    grid = (N // TILE,)
    out = pl.pallas_call(
        _fused_kernel,
        grid=grid,
        in_specs=[
            pl.BlockSpec((TILE, D_MODEL), lambda i: (i, 0)),
            pl.BlockSpec((WCAT_ROWS, D_MODEL), lambda i: (0, 0)),
            pl.BlockSpec((1, D_OUT), lambda i: (0, 0)),
            pl.BlockSpec((ER, D_OUT), lambda i: (0, 0)),
        ],
        out_specs=pl.BlockSpec((TILE, D_OUT), lambda i: (i, 0)),
        out_shape=jax.ShapeDtypeStruct((N, D_OUT), jnp.float32),
        compiler_params=pltpu.CompilerParams(
            dimension_semantics=("arbitrary",)),
    )(xf, w_cat, b2, b_cat)
    return out.reshape(B, S, D_OUT)


# speedup vs baseline: 1.0279x; 1.0279x over previous
import functools

import jax
import jax.numpy as jnp
from jax.experimental import pallas as pl
from jax.experimental.pallas import tpu as pltpu

D_MODEL = 2048
D_OUT = 2048
E = 8
R = 64
ER = E * R
SCALING = 128.0 / 64.0

TILE = 512
RW_PAD = 128
H_OFF = RW_PAD
B_OFF = RW_PAD + ER
WCAT_ROWS = RW_PAD + ER + D_OUT


def _pack_kernel(rwt_ref, a_ref, w_ref, lb_ref, wcat_ref, bcat_ref):
    wcat_ref[0:E, :] = rwt_ref[...].astype(jnp.bfloat16)
    wcat_ref[E:RW_PAD, :] = jnp.zeros((RW_PAD - E, D_MODEL), jnp.bfloat16)
    wcat_ref[H_OFF:B_OFF, :] = a_ref[...].astype(jnp.bfloat16)
    wcat_ref[B_OFF:, :] = w_ref[...].astype(jnp.bfloat16)
    lb = lb_ref[...]
    bcat_ref[...] = jnp.transpose(lb, (0, 2, 1)).astype(
        jnp.bfloat16).reshape(ER, D_OUT)


def _fused_kernel(xf_ref, wcat_ref, b_ref, bcat_ref, o_ref):
    xb = xf_ref[...].astype(jnp.bfloat16)

    big = jax.lax.dot_general(
        xb, wcat_ref[...], (((1,), (1,)), ((), ())),
        preferred_element_type=jnp.float32)

    logits = big[:, :E]
    h = big[:, H_OFF:B_OFF]
    base = big[:, B_OFF:]

    m = jnp.max(logits, axis=1, keepdims=True)
    p = jnp.exp(logits - m)
    eidx = jax.lax.broadcasted_iota(jnp.int32, (TILE, E), 1)

    v1 = jnp.max(p, axis=1, keepdims=True)
    i1 = jnp.min(jnp.where(p == v1, eidx, E), axis=1, keepdims=True)
    p2 = jnp.where(eidx == i1, -1.0, p)
    v2 = jnp.max(p2, axis=1, keepdims=True)
    i2 = jnp.min(jnp.where(p2 == v2, eidx, E), axis=1, keepdims=True)

    denom = v1 + v2
    g1 = (v1 / denom) * SCALING
    g2 = (v2 / denom) * SCALING

    ecol = jax.lax.broadcasted_iota(jnp.int32, (TILE, ER), 1) // R
    gates = jnp.where(ecol == i1, g1, 0.0) + jnp.where(ecol == i2, g2, 0.0)
    hw = (h * gates).astype(jnp.bfloat16)

    moe = jax.lax.dot_general(
        hw, bcat_ref[...], (((1,), (0,)), ((), ())),
        preferred_element_type=jnp.float32)

    o_ref[...] = base + moe + b_ref[...]


@functools.partial(jax.jit, static_argnames=())
def kernel(x, W_base, b_base, router_w, lora_A, lora_B):
    B, S, D = x.shape
    N = B * S
    xf = x.reshape(N, D)

    w_cat, b_cat = pl.pallas_call(
        _pack_kernel,
        out_shape=(
            jax.ShapeDtypeStruct((WCAT_ROWS, D_MODEL), jnp.bfloat16),
            jax.ShapeDtypeStruct((ER, D_OUT), jnp.bfloat16),
        ),
    )(router_w.T, lora_A.reshape(ER, D_MODEL), W_base, lora_B)
    b2 = b_base.reshape(1, D_OUT)

    grid = (N // TILE,)
    out = pl.pallas_call(
        _fused_kernel,
        grid=grid,
        in_specs=[
            pl.BlockSpec((TILE, D_MODEL), lambda i: (i, 0)),
            pl.BlockSpec((WCAT_ROWS, D_MODEL), lambda i: (0, 0)),
            pl.BlockSpec((1, D_OUT), lambda i: (0, 0)),
            pl.BlockSpec((ER, D_OUT), lambda i: (0, 0)),
        ],
        out_specs=pl.BlockSpec((TILE, D_OUT), lambda i: (i, 0)),
        out_shape=jax.ShapeDtypeStruct((N, D_OUT), jnp.float32),
        compiler_params=pltpu.CompilerParams(
            dimension_semantics=("arbitrary",)),
    )(xf, w_cat, b2, b_cat)
    return out.reshape(B, S, D_OUT)


# speedup vs baseline: 1.0504x; 1.0219x over previous
import functools

import jax
import jax.numpy as jnp
from jax.experimental import pallas as pl
from jax.experimental.pallas import tpu as pltpu

D_MODEL = 2048
D_OUT = 2048
E = 8
R = 64
ER = E * R
SCALING = 128.0 / 64.0

TILE = 512
B_OFF = ER
WCAT_ROWS = ER + D_OUT


def _pack_kernel(rwt_ref, a_ref, w_ref, lb_ref, wcat_ref, bcat_ref, rwo_ref):
    rwo_ref[...] = rwt_ref[...].astype(jnp.bfloat16)
    wcat_ref[0:B_OFF, :] = a_ref[...].astype(jnp.bfloat16)
    wcat_ref[B_OFF:, :] = w_ref[...].astype(jnp.bfloat16)
    lb = lb_ref[...]
    bcat_ref[...] = jnp.transpose(lb, (0, 2, 1)).astype(
        jnp.bfloat16).reshape(ER, D_OUT)


def _fused_kernel(xf_ref, wcat_ref, b_ref, bcat_ref, rw_ref, o_ref):
    xb = xf_ref[...].astype(jnp.bfloat16)

    logits = jax.lax.dot_general(
        xb, rw_ref[...], (((1,), (1,)), ((), ())),
        preferred_element_type=jnp.float32)

    big = jax.lax.dot_general(
        xb, wcat_ref[...], (((1,), (1,)), ((), ())),
        preferred_element_type=jnp.float32)

    h = big[:, :B_OFF]
    base = big[:, B_OFF:]

    m = jnp.max(logits, axis=1, keepdims=True)
    p = jnp.exp(logits - m)
    eidx = jax.lax.broadcasted_iota(jnp.int32, (TILE, E), 1)

    v1 = jnp.max(p, axis=1, keepdims=True)
    i1 = jnp.min(jnp.where(p == v1, eidx, E), axis=1, keepdims=True)
    p2 = jnp.where(eidx == i1, -1.0, p)
    v2 = jnp.max(p2, axis=1, keepdims=True)
    i2 = jnp.min(jnp.where(p2 == v2, eidx, E), axis=1, keepdims=True)

    denom = v1 + v2
    g1 = (v1 / denom) * SCALING
    g2 = (v2 / denom) * SCALING

    ecol = jax.lax.broadcasted_iota(jnp.int32, (TILE, ER), 1) // R
    gates = jnp.where(ecol == i1, g1, 0.0) + jnp.where(ecol == i2, g2, 0.0)
    hw = (h * gates).astype(jnp.bfloat16)

    moe = jax.lax.dot_general(
        hw, bcat_ref[...], (((1,), (0,)), ((), ())),
        preferred_element_type=jnp.float32)

    o_ref[...] = base + moe + b_ref[...]


@functools.partial(jax.jit, static_argnames=())
def kernel(x, W_base, b_base, router_w, lora_A, lora_B):
    B, S, D = x.shape
    N = B * S
    xf = x.reshape(N, D)

    w_cat, b_cat, rw_bf = pl.pallas_call(
        _pack_kernel,
        out_shape=(
            jax.ShapeDtypeStruct((WCAT_ROWS, D_MODEL), jnp.bfloat16),
            jax.ShapeDtypeStruct((ER, D_OUT), jnp.bfloat16),
            jax.ShapeDtypeStruct((E, D_MODEL), jnp.bfloat16),
        ),
    )(router_w.T, lora_A.reshape(ER, D_MODEL), W_base, lora_B)
    b2 = b_base.reshape(1, D_OUT)

    grid = (N // TILE,)
    out = pl.pallas_call(
        _fused_kernel,
        grid=grid,
        in_specs=[
            pl.BlockSpec((TILE, D_MODEL), lambda i: (i, 0)),
            pl.BlockSpec((WCAT_ROWS, D_MODEL), lambda i: (0, 0)),
            pl.BlockSpec((1, D_OUT), lambda i: (0, 0)),
            pl.BlockSpec((ER, D_OUT), lambda i: (0, 0)),
            pl.BlockSpec((E, D_MODEL), lambda i: (0, 0)),
        ],
        out_specs=pl.BlockSpec((TILE, D_OUT), lambda i: (i, 0)),
        out_shape=jax.ShapeDtypeStruct((N, D_OUT), jnp.float32),
        compiler_params=pltpu.CompilerParams(
            dimension_semantics=("arbitrary",)),
    )(xf, w_cat, b2, b_cat, rw_bf)
    return out.reshape(B, S, D_OUT)
